# baseline (speedup 1.0000x reference)
import functools

import jax
import jax.numpy as jnp
from jax.experimental import pallas as pl
from jax.experimental.pallas import tpu as pltpu

B, S, D, E = 2, 2048, 1024, 8
T = B * S
EP = 128
M = 512
NB = T // M


def _gate_body(x_ref, wg_ref, gate_ref, selp_ref, psum_ref, cnt_ref):
    xb = x_ref[...]
    wg = wg_ref[...]
    logits = jnp.dot(xb, wg, preferred_element_type=jnp.float32)
    lane = jax.lax.broadcasted_iota(jnp.int32, (M, EP), 1)
    valid = lane < E
    neg = jnp.full_like(logits, -jnp.inf)
    logit_m = jnp.where(valid, logits, neg)
    mx = jnp.max(logit_m, axis=-1, keepdims=True)
    ex = jnp.where(valid, jnp.exp(logit_m - mx), 0.0)
    den = jnp.sum(ex, axis=-1, keepdims=True)
    probs = ex / den
    gate = jnp.argmax(logit_m, axis=-1).astype(jnp.int32)
    selp = jnp.max(probs, axis=-1)
    gate_ref[0, 0, :] = gate
    selp_ref[0, 0, :] = selp
    psum_ref[0, 0, :] = jnp.sum(probs, axis=0)
    onehot = jnp.where(lane == gate[:, None], 1.0, 0.0)
    cnt_ref[0, 0, :] = jnp.sum(onehot, axis=0)


def _expert_body(gate_ref, selp_ref, x_ref, w_ref, b_ref, out_ref):
    e = pl.program_id(1)
    xb = x_ref[...]
    w = w_ref[0]
    y = jnp.dot(xb, w, preferred_element_type=jnp.float32) + b_ref[0, 0, :][None, :]
    gate = gate_ref[0, 0, :]
    selp = selp_ref[0, 0, :]
    scale = jnp.where(gate == e, selp, 0.0)
    contrib = y * scale[:, None]
    @pl.when(e == 0)
    def _():
        out_ref[...] = contrib
    @pl.when(e > 0)
    def _():
        out_ref[...] += contrib


def kernel(x, attention_mask, W_gate, W_experts, b_experts):
    del attention_mask
    xf = x.reshape(T, D)
    wg_pad = jnp.zeros((D, EP), jnp.float32).at[:, :E].set(W_gate)

    gate3, selp3, psum3, cnt3 = pl.pallas_call(
        _gate_body,
        grid=(NB,),
        in_specs=[
            pl.BlockSpec((M, D), lambda b: (b, 0)),
            pl.BlockSpec((D, EP), lambda b: (0, 0)),
        ],
        out_specs=[
            pl.BlockSpec((1, 1, M), lambda b: (b, 0, 0)),
            pl.BlockSpec((1, 1, M), lambda b: (b, 0, 0)),
            pl.BlockSpec((1, 1, EP), lambda b: (b, 0, 0)),
            pl.BlockSpec((1, 1, EP), lambda b: (b, 0, 0)),
        ],
        out_shape=[
            jax.ShapeDtypeStruct((NB, 1, M), jnp.int32),
            jax.ShapeDtypeStruct((NB, 1, M), jnp.float32),
            jax.ShapeDtypeStruct((NB, 1, EP), jnp.float32),
            jax.ShapeDtypeStruct((NB, 1, EP), jnp.float32),
        ],
    )(xf, wg_pad)

    out = pl.pallas_call(
        _expert_body,
        grid=(NB, E),
        in_specs=[
            pl.BlockSpec((1, 1, M), lambda b, e: (b, 0, 0)),
            pl.BlockSpec((1, 1, M), lambda b, e: (b, 0, 0)),
            pl.BlockSpec((M, D), lambda b, e: (b, 0)),
            pl.BlockSpec((1, D, D), lambda b, e: (e, 0, 0)),
            pl.BlockSpec((1, 1, D), lambda b, e: (e, 0, 0)),
        ],
        out_specs=pl.BlockSpec((M, D), lambda b, e: (b, 0)),
        out_shape=jax.ShapeDtypeStruct((T, D), jnp.float32),
    )(gate3, selp3, xf.astype(jnp.bfloat16), W_experts.astype(jnp.bfloat16),
      b_experts.reshape(E, 1, D))

    psum = jnp.sum(psum3[:, 0, :E], axis=0)
    counts_f = jnp.sum(cnt3[:, 0, :E], axis=0)
    P = psum / jnp.float32(T)
    f = counts_f / jnp.sum(counts_f)
    balance_loss = jnp.float32(E) * jnp.sum(P * f)
    gate_load = counts_f.astype(jnp.int32)
    return out.reshape(B, S, D), balance_loss, gate_load


# speedup vs baseline: 1.3610x; 1.3610x over previous
import functools

import jax
import jax.numpy as jnp
from jax import lax
from jax.experimental import pallas as pl
from jax.experimental.pallas import tpu as pltpu
from jax.experimental.pallas import tpu_sc as plsc

B, S, D, E = 2, 2048, 1024, 8
T = B * S
EP = 128
M = 512
NB = T // M
WMAX = NB + E - 1
XCOL = D + 128

NW = 32
TPW = T // NW
CS = 32
NCH = TPW // CS


def _gate_body(x_ref, wg_ref, gate_ref, rank_ref, psum_ref, cnt_ref, xsc_ref,
               run_ref):
    b = pl.program_id(0)

    @pl.when(b == 0)
    def _():
        run_ref[...] = jnp.zeros_like(run_ref)

    xb = x_ref[...]
    wg = wg_ref[...]
    logits = jnp.dot(xb, wg, preferred_element_type=jnp.float32)
    lane = lax.broadcasted_iota(jnp.int32, (M, EP), 1)
    valid = lane < E
    neg = jnp.full_like(logits, -jnp.inf)
    logit_m = jnp.where(valid, logits, neg)
    mx = jnp.max(logit_m, axis=-1, keepdims=True)
    ex = jnp.where(valid, jnp.exp(logit_m - mx), 0.0)
    den = jnp.sum(ex, axis=-1, keepdims=True)
    probs = ex / den
    gate = jnp.argmax(logit_m, axis=-1).astype(jnp.int32)
    selp = jnp.max(probs, axis=-1)

    onehot = jnp.where(lane == gate[:, None], 1.0, 0.0)
    ri = lax.broadcasted_iota(jnp.int32, (M, M), 0)
    cj = lax.broadcasted_iota(jnp.int32, (M, M), 1)
    lower = jnp.where(cj < ri, 1.0, 0.0)
    cum_excl = jnp.dot(lower, onehot, preferred_element_type=jnp.float32)
    local_rank = jnp.sum(cum_excl * onehot, axis=1)
    carry = jnp.sum(run_ref[...] * onehot, axis=1)
    rank = (local_rank + carry).astype(jnp.int32)

    gate_ref[0, 0, :] = gate
    rank_ref[0, 0, :] = rank
    psum_ref[0, 0, :] = jnp.sum(probs, axis=0)
    cnt_ref[0, 0, :] = jnp.sum(onehot, axis=0)
    run_ref[...] = run_ref[...] + jnp.sum(onehot, axis=0, keepdims=True)

    xsc_ref[:, :D] = xb * selp[:, None]
    lane2 = lax.broadcasted_iota(jnp.int32, (M, XCOL - D), 1)
    xsc_ref[:, D:] = jnp.where(lane2 == 0, selp[:, None], 0.0)


def _pos_chunks(gate_hbm, rank_hbm, off_hbm, g_v, r_v, o_v, pos_v, base):
    pltpu.sync_copy(gate_hbm.at[pl.ds(base, TPW)], g_v)
    pltpu.sync_copy(rank_hbm.at[pl.ds(base, TPW)], r_v)
    pltpu.sync_copy(off_hbm, o_v)
    for c in range(TPW // 16):
        g16 = g_v[pl.ds(c * 16, 16)]
        off16 = plsc.load_gather(o_v, [g16])
        j, k = divmod(c * 16, CS)
        pos_v[j, pl.ds(k, 16)] = off16 + r_v[pl.ds(c * 16, 16)]


def _dispatch_body(gate_hbm, rank_hbm, off_hbm, xsc_hbm, xs_hbm,
                   g_v, r_v, o_v, pos_v, rows_v, sem):
    wid = lax.axis_index("s") * 2 + lax.axis_index("c")
    base = wid * TPW
    _pos_chunks(gate_hbm, rank_hbm, off_hbm, g_v, r_v, o_v, pos_v, base)
    for j in range(NCH):
        pltpu.sync_copy(xsc_hbm.at[pl.ds(base + j * CS, CS)], rows_v)
        pltpu.async_copy(rows_v, xs_hbm.at[pos_v.at[j]], sem).wait()


def _combine_body(gate_hbm, rank_hbm, off_hbm, ys_hbm, out_hbm,
                  g_v, r_v, o_v, pos_v, rows_v, sem):
    wid = lax.axis_index("s") * 2 + lax.axis_index("c")
    base = wid * TPW
    _pos_chunks(gate_hbm, rank_hbm, off_hbm, g_v, r_v, o_v, pos_v, base)
    for j in range(NCH):
        pltpu.async_copy(ys_hbm.at[pos_v.at[j]], rows_v, sem).wait()
        pltpu.sync_copy(rows_v, out_hbm.at[pl.ds(base + j * CS, CS)])


@functools.cache
def _sc_kernels():
    mesh = plsc.VectorSubcoreMesh(core_axis_name="c", subcore_axis_name="s")
    params = pltpu.CompilerParams(needs_layout_passes=False)
    dispatch = pl.kernel(
        _dispatch_body, mesh=mesh, compiler_params=params,
        out_type=jax.ShapeDtypeStruct((T, XCOL), jnp.float32),
        scratch_types=[
            pltpu.VMEM((TPW,), jnp.int32),
            pltpu.VMEM((TPW,), jnp.int32),
            pltpu.VMEM((16,), jnp.int32),
            pltpu.VMEM((NCH, CS), jnp.int32),
            pltpu.VMEM((CS, XCOL), jnp.float32),
            pltpu.SemaphoreType.DMA,
        ],
    )
    combine = pl.kernel(
        _combine_body, mesh=mesh, compiler_params=params,
        out_type=jax.ShapeDtypeStruct((T, D), jnp.float32),
        scratch_types=[
            pltpu.VMEM((TPW,), jnp.int32),
            pltpu.VMEM((TPW,), jnp.int32),
            pltpu.VMEM((16,), jnp.int32),
            pltpu.VMEM((NCH, CS), jnp.int32),
            pltpu.VMEM((CS, D), jnp.float32),
            pltpu.SemaphoreType.DMA,
        ],
    )
    return dispatch, combine


def _moe_body(blk_s, eid_s, rs_s, re_s, xs_ref, w_ref, b_ref, ys_ref):
    del eid_s
    w = pl.program_id(0)
    blk = blk_s[w]
    prev_blk = blk_s[jnp.maximum(w - 1, 0)]
    first = jnp.logical_or(w == 0, blk != prev_blk)
    xb = xs_ref[...]
    y = jnp.dot(xb[:, :D], w_ref[0], preferred_element_type=jnp.float32)
    y = y + xb[:, D:D + 1] * b_ref[0, 0, :][None, :]
    jg = blk * M + lax.broadcasted_iota(jnp.int32, (M, 1), 0)
    mask = jnp.logical_and(jg >= rs_s[w], jg < re_s[w])
    contrib = jnp.where(mask, y, 0.0)
    ys_ref[...] = jnp.where(first, contrib, ys_ref[...] + contrib)


def kernel(x, attention_mask, W_gate, W_experts, b_experts):
    del attention_mask
    xf = x.reshape(T, D)
    wg_pad = jnp.zeros((D, EP), jnp.float32).at[:, :E].set(W_gate)

    gate3, rank3, psum3, cnt3, xsc = pl.pallas_call(
        _gate_body,
        grid=(NB,),
        in_specs=[
            pl.BlockSpec((M, D), lambda b: (b, 0)),
            pl.BlockSpec((D, EP), lambda b: (0, 0)),
        ],
        out_specs=[
            pl.BlockSpec((1, 1, M), lambda b: (b, 0, 0)),
            pl.BlockSpec((1, 1, M), lambda b: (b, 0, 0)),
            pl.BlockSpec((1, 1, EP), lambda b: (b, 0, 0)),
            pl.BlockSpec((1, 1, EP), lambda b: (b, 0, 0)),
            pl.BlockSpec((M, XCOL), lambda b: (b, 0)),
        ],
        out_shape=[
            jax.ShapeDtypeStruct((NB, 1, M), jnp.int32),
            jax.ShapeDtypeStruct((NB, 1, M), jnp.int32),
            jax.ShapeDtypeStruct((NB, 1, EP), jnp.float32),
            jax.ShapeDtypeStruct((NB, 1, EP), jnp.float32),
            jax.ShapeDtypeStruct((T, XCOL), jnp.float32),
        ],
        scratch_shapes=[pltpu.VMEM((1, EP), jnp.float32)],
    )(xf, wg_pad)

    gate = gate3.reshape(T)
    rank = rank3.reshape(T)
    counts_f = jnp.sum(cnt3[:, 0, :E], axis=0)
    counts = counts_f.astype(jnp.int32)
    off = jnp.concatenate([jnp.zeros((1,), jnp.int32), jnp.cumsum(counts)])
    off_pad = jnp.concatenate([off, jnp.full((16 - E - 1,), T, jnp.int32)])

    _dispatch, _combine = _sc_kernels()
    xs = _dispatch(gate, rank, off_pad, xsc)

    bb = jnp.arange(NB, dtype=jnp.int32)[:, None]
    ee = jnp.arange(E, dtype=jnp.int32)[None, :]
    seg_s = jnp.maximum(off[ee], bb * M)
    seg_e = jnp.minimum(off[ee + 1], (bb + 1) * M)
    active = (seg_e > seg_s).reshape(-1)
    cpos = jnp.cumsum(active.astype(jnp.int32)) - 1
    slot = jnp.where(active, cpos, WMAX)
    bb_f = jnp.broadcast_to(bb, (NB, E)).reshape(-1)
    ee_f = jnp.broadcast_to(ee, (NB, E)).reshape(-1)

    def scat(init, vals):
        return jnp.full((WMAX + 1,), init, jnp.int32).at[slot].set(vals)[:WMAX]

    blk_l = scat(NB - 1, bb_f)
    eid_l = scat(0, ee_f)
    rs_l = scat(0, seg_s.reshape(-1))
    re_l = scat(0, seg_e.reshape(-1))

    grid_spec = pltpu.PrefetchScalarGridSpec(
        num_scalar_prefetch=4,
        grid=(WMAX,),
        in_specs=[
            pl.BlockSpec((M, XCOL), lambda w, blk, eid, rs, re: (blk[w], 0)),
            pl.BlockSpec((1, D, D), lambda w, blk, eid, rs, re: (eid[w], 0, 0)),
            pl.BlockSpec((1, 1, D), lambda w, blk, eid, rs, re: (eid[w], 0, 0)),
        ],
        out_specs=pl.BlockSpec((M, D), lambda w, blk, eid, rs, re: (blk[w], 0)),
    )
    ys = pl.pallas_call(
        _moe_body,
        grid_spec=grid_spec,
        out_shape=jax.ShapeDtypeStruct((T, D), jnp.float32),
    )(blk_l, eid_l, rs_l, re_l, xs, W_experts, b_experts.reshape(E, 1, D))

    out = _combine(gate, rank, off_pad, ys)

    psum = jnp.sum(psum3[:, 0, :E], axis=0)
    P = psum / jnp.float32(T)
    f = counts_f / jnp.sum(counts_f)
    balance_loss = jnp.float32(E) * jnp.sum(P * f)
    return out.reshape(B, S, D), balance_loss, counts


if __name__ == "__main__":
    pass
